# baseline (speedup 1.0000x reference)
import functools

import jax
import jax.numpy as jnp
from jax.experimental import pallas as pl
from jax.experimental.pallas import tpu as pltpu

N = 2048
IN_F = 128
OUT_F = 32
AUG = 64
ALPHA = 0.2
CHUNKS = (1024, 512, 256, 256)
OFFS = (0, 1024, 1536, 1792)

_CONTRACT_LAST = (((1,), (1,)), ((), ()))


def _gat_kernel(x_ref, adj_hbm, w_ref, a_ref, out_ref,
                haug_ref, s1_ref, s2_ref, *bufs_and_sems):
    bufs = bufs_and_sems[:len(CHUNKS)]
    sems = bufs_and_sems[len(CHUNKS):]

    copies = [
        pltpu.make_async_copy(
            adj_hbm.at[pl.ds(OFFS[k], CHUNKS[k]), :], bufs[k], sems[k])
        for k in range(len(CHUNKS))
    ]
    copies[0].start()

    h = jnp.dot(x_ref[...], w_ref[...],
                preferred_element_type=jnp.float32,
                precision=jax.lax.Precision.HIGHEST)
    ones = jnp.ones((N, 1), dtype=jnp.float32)
    zeros = jnp.zeros((N, AUG - OUT_F - 1), dtype=jnp.float32)
    haug_ref[...] = jnp.concatenate([h, ones, zeros], axis=1).astype(jnp.bfloat16)
    scale = -1.4426950408889634
    s1_ref[...] = jax.lax.dot_general(
        h, scale * a_ref[0:1, :OUT_F], _CONTRACT_LAST,
        preferred_element_type=jnp.float32,
        precision=jax.lax.Precision.HIGHEST)
    s2_ref[...] = jax.lax.dot_general(
        scale * a_ref[0:1, OUT_F:], h, _CONTRACT_LAST,
        preferred_element_type=jnp.float32,
        precision=jax.lax.Precision.HIGHEST)

    for k, (off, ch) in enumerate(zip(OFFS, CHUNKS)):
        copies[k].wait()
        if k + 1 < len(CHUNKS):
            copies[k + 1].start()
        adj_blk = bufs[k][...]
        s1b = s1_ref[pl.ds(off, ch), :]
        t = s1b + s2_ref[...]
        arg = jnp.minimum(t, ALPHA * t)
        ee = jnp.where(adj_blk != 0, jnp.exp2(arg), 0.0).astype(jnp.bfloat16)
        hp_aug = jnp.dot(ee, haug_ref[...],
                         preferred_element_type=jnp.float32)
        hp = hp_aug[:, :OUT_F] / hp_aug[:, OUT_F:OUT_F + 1]
        out_ref[pl.ds(off, ch), :] = jnp.where(
            hp > 0, hp, jnp.exp(hp) - 1.0)


@functools.partial(jax.jit, static_argnames=())
def kernel(input, adj, W, a):
    return pl.pallas_call(
        _gat_kernel,
        in_specs=[
            pl.BlockSpec(memory_space=pltpu.VMEM),
            pl.BlockSpec(memory_space=pl.ANY),
            pl.BlockSpec(memory_space=pltpu.VMEM),
            pl.BlockSpec(memory_space=pltpu.VMEM),
        ],
        out_specs=pl.BlockSpec(memory_space=pltpu.VMEM),
        out_shape=jax.ShapeDtypeStruct((N, OUT_F), jnp.float32),
        scratch_shapes=[
            pltpu.VMEM((N, AUG), jnp.bfloat16),
            pltpu.VMEM((N, 1), jnp.float32),
            pltpu.VMEM((1, N), jnp.float32),
            *[pltpu.VMEM((ch, N), jnp.int32) for ch in CHUNKS],
            *[pltpu.SemaphoreType.DMA for _ in CHUNKS],
        ],
    )(input, adj, W, a)


# speedup vs baseline: 1.0810x; 1.0810x over previous
import functools

import jax
import jax.numpy as jnp
from jax.experimental import pallas as pl
from jax.experimental.pallas import tpu as pltpu

N = 2048
IN_F = 128
OUT_F = 32
AUG = 64
ALPHA = 0.2
CHUNKS = (1024, 1024)
OFFS = (0, 1024)

_CONTRACT_LAST = (((1,), (1,)), ((), ()))


def _gat_kernel(x_ref, adj_hbm, w_ref, a_ref, out_ref,
                haug_ref, s1_ref, s2_ref, *bufs_and_sems):
    bufs = bufs_and_sems[:len(CHUNKS)]
    sems = bufs_and_sems[len(CHUNKS):]

    copies = [
        pltpu.make_async_copy(
            adj_hbm.at[pl.ds(OFFS[k], CHUNKS[k]), :], bufs[k], sems[k])
        for k in range(len(CHUNKS))
    ]
    for c in copies:
        c.start()

    h = jnp.dot(x_ref[...], w_ref[...],
                preferred_element_type=jnp.float32,
                precision=jax.lax.Precision.HIGHEST)
    ones = jnp.ones((N, 1), dtype=jnp.float32)
    zeros = jnp.zeros((N, AUG - OUT_F - 1), dtype=jnp.float32)
    haug_ref[...] = jnp.concatenate([h, ones, zeros], axis=1).astype(jnp.bfloat16)
    scale = -1.4426950408889634
    s1_ref[...] = jax.lax.dot_general(
        h, scale * a_ref[0:1, :OUT_F], _CONTRACT_LAST,
        preferred_element_type=jnp.float32,
        precision=jax.lax.Precision.HIGHEST)
    s2_ref[...] = jax.lax.dot_general(
        scale * a_ref[0:1, OUT_F:], h, _CONTRACT_LAST,
        preferred_element_type=jnp.float32,
        precision=jax.lax.Precision.HIGHEST)

    for k, (off, ch) in enumerate(zip(OFFS, CHUNKS)):
        copies[k].wait()
        adj_blk = bufs[k][...]
        s1b = s1_ref[pl.ds(off, ch), :]
        t = s1b + s2_ref[...]
        arg = jnp.minimum(t, ALPHA * t)
        ee = jnp.where(adj_blk != 0, jnp.exp2(arg), 0.0).astype(jnp.bfloat16)
        hp_aug = jnp.dot(ee, haug_ref[...],
                         preferred_element_type=jnp.float32)
        hp = hp_aug[:, :OUT_F] / hp_aug[:, OUT_F:OUT_F + 1]
        out_ref[pl.ds(off, ch), :] = jnp.where(
            hp > 0, hp, jnp.exp(hp) - 1.0)


@functools.partial(jax.jit, static_argnames=())
def kernel(input, adj, W, a):
    return pl.pallas_call(
        _gat_kernel,
        in_specs=[
            pl.BlockSpec(memory_space=pltpu.VMEM),
            pl.BlockSpec(memory_space=pl.ANY),
            pl.BlockSpec(memory_space=pltpu.VMEM),
            pl.BlockSpec(memory_space=pltpu.VMEM),
        ],
        out_specs=pl.BlockSpec(memory_space=pltpu.VMEM),
        out_shape=jax.ShapeDtypeStruct((N, OUT_F), jnp.float32),
        scratch_shapes=[
            pltpu.VMEM((N, AUG), jnp.bfloat16),
            pltpu.VMEM((N, 1), jnp.float32),
            pltpu.VMEM((1, N), jnp.float32),
            *[pltpu.VMEM((ch, N), jnp.int32) for ch in CHUNKS],
            *[pltpu.SemaphoreType.DMA for _ in CHUNKS],
        ],
    )(input, adj, W, a)
